# baseline (speedup 1.0000x reference)
import functools

import jax
import jax.numpy as jnp
from jax import lax
from jax.experimental import pallas as pl
from jax.experimental.pallas import tpu as pltpu
from jax.experimental.pallas import tpu_sc as plsc

_B, _N, _D = 4, 256, 64
_NC, _NS = 2, 16
_NW = _NC * _NS
_ROWS = _B * _N // _NW
_CH = _D // 16


def _precompute_body(pos_ref, vel_ref, mask_ref, idex_ref,
                     dp_ref, rs_ref, m_ref, cnt_ref):
    px = pos_ref[0, :, 0]
    py = pos_ref[0, :, 1]
    dx = px[:, None] - px[None, :]
    dy = py[:, None] - py[None, :]
    dp_ref[0] = jnp.sqrt(dx * dx + dy * dy) + 1e-6
    vx = vel_ref[0, :, 0]
    vy = vel_ref[0, :, 1]
    wx = vx[:, None] - vx[None, :]
    wy = vy[:, None] - vy[None, :]
    rs_ref[0] = jnp.sqrt(wx * wx + wy * wy)
    mask = mask_ref[0]
    m_ref[0] = jnp.where(mask != 0, idex_ref[0], _N)
    cnt_ref[0] = jnp.sum(mask.astype(jnp.float32), axis=-1, keepdims=True)


def _precompute(pos, vel, mask, idex):
    grid = (_B,)
    bs3 = lambda d2: pl.BlockSpec((1, _N, d2), lambda b: (b, 0, 0))
    return pl.pallas_call(
        _precompute_body,
        grid=grid,
        in_specs=[bs3(2), bs3(2), bs3(_N), bs3(_N)],
        out_specs=[bs3(_N), bs3(_N), bs3(_N), bs3(1)],
        out_shape=[
            jax.ShapeDtypeStruct((_B, _N, _N), jnp.float32),
            jax.ShapeDtypeStruct((_B, _N, _N), jnp.float32),
            jax.ShapeDtypeStruct((_B, _N, _N), jnp.int32),
            jax.ShapeDtypeStruct((_B, _N, 1), jnp.float32),
        ],
    )(pos, vel, mask, idex)


def _tbase_body(h_ref, pos_ref, W1_ref, be1_ref, t_ref, base_ref):
    hb = h_ref[0]
    t = jnp.dot(hb, W1_ref[:_D, :], preferred_element_type=jnp.float32)
    px = pos_ref[0, :, 0]
    py = pos_ref[0, :, 1]
    w64 = W1_ref[_D, :]
    w65 = W1_ref[_D + 1, :]
    pw = px[:, None] * w64[None, :] + py[:, None] * w65[None, :]
    t_ref[0] = t + pw
    base_ref[0] = be1_ref[0, :][None, :] - pw


def _tbase(h, pos, We1, be1):
    grid = (_B,)
    return pl.pallas_call(
        _tbase_body,
        grid=grid,
        in_specs=[
            pl.BlockSpec((1, _N, _D), lambda b: (b, 0, 0)),
            pl.BlockSpec((1, _N, 2), lambda b: (b, 0, 0)),
            pl.BlockSpec((_D + 4, _D), lambda b: (0, 0)),
            pl.BlockSpec((1, _D), lambda b: (0, 0)),
        ],
        out_specs=[
            pl.BlockSpec((1, _N, _D), lambda b: (b, 0, 0)),
            pl.BlockSpec((1, _N, _D), lambda b: (b, 0, 0)),
        ],
        out_shape=[
            jax.ShapeDtypeStruct((_B, _N, _D), jnp.float32),
            jax.ShapeDtypeStruct((_B, _N, _D), jnp.float32),
        ],
    )(h, pos, We1, be1.reshape(1, _D))


_GDN = lax.GatherDimensionNumbers(offset_dims=(), collapsed_slice_dims=(0,),
                                  start_index_map=(0,))


def _lane_bcast(v, idx):
    return lax.gather(v, idx, _GDN, (1,),
                      mode=lax.GatherScatterMode.PROMISE_IN_BOUNDS)


def _edge_body(t_hbm, base_hbm, m_hbm, dp_hbm, rs_hbm, w_hbm,
               S_hbm,
               t_v, base_v, m_v, dp_v, rs_v, w_v, S_v):
    wid = lax.axis_index("s") * _NC + lax.axis_index("c")
    r0 = wid * _ROWS
    b = r0 // _N
    pltpu.sync_copy(t_hbm.at[b], t_v.at[pl.ds(0, _N * _D)])
    pltpu.sync_copy(base_hbm.at[pl.ds(r0, _ROWS)], base_v)
    pltpu.sync_copy(m_hbm.at[pl.ds(r0, _ROWS)], m_v)
    pltpu.sync_copy(dp_hbm.at[pl.ds(r0 * _N, _ROWS * _N)],
                    dp_v.at[pl.ds(0, _ROWS * _N)])
    pltpu.sync_copy(rs_hbm.at[pl.ds(r0 * _N, _ROWS * _N)],
                    rs_v.at[pl.ds(0, _ROWS * _N)])
    pltpu.sync_copy(w_hbm, w_v)
    for c in range(_CH):
        t_v[pl.ds(_N * _D + 16 * c, 16)] = jnp.full((16,), -1e30, jnp.float32)
    dp_v[pl.ds(_ROWS * _N, 16)] = jnp.zeros((16,), jnp.float32)
    rs_v[pl.ds(_ROWS * _N, 16)] = jnp.zeros((16,), jnp.float32)

    cols = [lax.iota(jnp.int32, 16) + 16 * c for c in range(_CH)]
    eidx = [jnp.full((16, 1), e, jnp.int32) for e in range(16)]
    wd = [w_v[0, pl.ds(16 * c, 16)] for c in range(_CH)]
    wrs = [w_v[1, pl.ds(16 * c, 16)] for c in range(_CH)]
    zeros8 = tuple(jnp.zeros((16,), jnp.float32) for _ in range(2 * _CH))

    @plsc.parallel_loop(0, _ROWS)
    def row_body(ii):
        bse = [base_v[ii, pl.ds(16 * c, 16)] for c in range(_CH)]
        ii16 = jnp.full((16,), ii * _N, jnp.int32)

        @plsc.parallel_loop(0, _N // 16, carry=zeros8)
        def chunk_body(kc, S):
            k0 = kc * 16
            j16 = m_v[ii, pl.ds(k0, 16)]
            d16 = plsc.load_gather(dp_v, [ii16 + j16])
            rs16 = plsc.load_gather(rs_v, [ii16 + j16])
            jm16 = j16 * _D
            S = list(S)
            for e in range(16):
                jb = _lane_bcast(jm16, eidx[e])
                db = _lane_bcast(d16, eidx[e])
                rsb = _lane_bcast(rs16, eidx[e])
                g = (e & 1) * _CH
                for c in range(_CH):
                    tg = plsc.load_gather(t_v, [jb + cols[c]])
                    a = db * wd[c] + (rsb * wrs[c] + bse[c])
                    S[g + c] = S[g + c] + jnp.maximum(tg + a, 0.0)
            return tuple(S)

        S = chunk_body
        for c in range(_CH):
            S_v[ii, pl.ds(16 * c, 16)] = S[c] + S[_CH + c]
    pltpu.sync_copy(S_v, S_hbm.at[pl.ds(r0, _ROWS)])


def _edge_pass(t_flat, base, m, dpr, rspr, wdrs):
    mesh = plsc.VectorSubcoreMesh(core_axis_name="c", subcore_axis_name="s",
                                  num_cores=_NC, num_subcores=_NS)
    f = pl.kernel(
        _edge_body,
        mesh=mesh,
        compiler_params=pltpu.CompilerParams(use_tc_tiling_on_sc=False,
                                             needs_layout_passes=False),
        out_type=jax.ShapeDtypeStruct((_B * _N, _D), jnp.float32),
        scratch_types=[
            pltpu.VMEM(((_N + 1) * _D,), jnp.float32),
            pltpu.VMEM((_ROWS, _D), jnp.float32),
            pltpu.VMEM((_ROWS, _N), jnp.int32),
            pltpu.VMEM((_ROWS * _N + 16,), jnp.float32),
            pltpu.VMEM((_ROWS * _N + 16,), jnp.float32),
            pltpu.VMEM((2, _D), jnp.float32),
            pltpu.VMEM((_ROWS, _D), jnp.float32),
        ],
    )
    return f(t_flat, base, m, dpr, rspr, wdrs)


def _node_body(h_ref, S_ref, cnt_ref, crowd_ref, W2_ref, be2_ref,
               Wn_ref, bn_ref, gb_ref, out_ref):
    hb = h_ref[0]
    S = S_ref[0]
    cnt = cnt_ref[0]
    agg = jnp.dot(S, W2_ref[...], preferred_element_type=jnp.float32)
    agg = (agg + cnt * be2_ref[0, :][None, :]) / (cnt + 1e-6)
    c = crowd_ref[0]
    mu = jnp.mean(c, axis=-1, keepdims=True)
    var = jnp.mean((c - mu) ** 2, axis=-1, keepdims=True)
    c1 = (c - mu) / jnp.sqrt(var + 1e-5) * gb_ref[0, :][None, :] + gb_ref[1, :][None, :]
    acc = jnp.dot(hb, Wn_ref[:_D, :], preferred_element_type=jnp.float32)
    acc += jnp.dot(agg, Wn_ref[_D:2 * _D, :], preferred_element_type=jnp.float32)
    for cc in range(5):
        acc += c1[:, cc:cc + 1] * Wn_ref[2 * _D + cc, :][None, :]
    out_ref[0] = jnp.maximum(acc + bn_ref[0, :][None, :], 0.0)


def _node_update(h, S, cnt, crowd, p):
    grid = (_B,)
    crowd_dim = crowd.shape[-1]
    gb = jnp.stack([p['g'], p['b']], axis=0)
    return pl.pallas_call(
        _node_body,
        grid=grid,
        in_specs=[
            pl.BlockSpec((1, _N, _D), lambda b: (b, 0, 0)),
            pl.BlockSpec((1, _N, _D), lambda b: (b, 0, 0)),
            pl.BlockSpec((1, _N, 1), lambda b: (b, 0, 0)),
            pl.BlockSpec((1, _N, crowd_dim), lambda b: (b, 0, 0)),
            pl.BlockSpec((_D, _D), lambda b: (0, 0)),
            pl.BlockSpec((1, _D), lambda b: (0, 0)),
            pl.BlockSpec((2 * _D + crowd_dim, _D), lambda b: (0, 0)),
            pl.BlockSpec((1, _D), lambda b: (0, 0)),
            pl.BlockSpec((2, crowd_dim), lambda b: (0, 0)),
        ],
        out_specs=pl.BlockSpec((1, _N, _D), lambda b: (b, 0, 0)),
        out_shape=jax.ShapeDtypeStruct((_B, _N, _D), jnp.float32),
    )(h, S, cnt, crowd, p['We2'], p['be2'].reshape(1, _D),
      p['Wn'], p['bn'].reshape(1, _D), gb)


def kernel(h, pos, vel, acc, crowd, mask, idex, hist, params):
    del acc, hist
    dpair, rspair, m, cnt = _precompute(pos, vel, mask, idex)
    m2 = m.reshape(_B * _N, _N)
    dp2 = dpair.reshape(_B * _N * _N)
    rs2 = rspair.reshape(_B * _N * _N)
    x = h
    for p in params:
        t, base = _tbase(x, pos, p['We1'], p['be1'])
        wdrs = jnp.stack([p['We1'][_D + 2], p['We1'][_D + 3]], axis=0)
        S = _edge_pass(t.reshape(_B, _N * _D), base.reshape(_B * _N, _D),
                       m2, dp2, rs2, wdrs)
        x = _node_update(x, S.reshape(_B, _N, _D), cnt, crowd, p)
    return x


# speedup vs baseline: 3.3277x; 3.3277x over previous
import functools

import jax
import jax.numpy as jnp
from jax import lax
from jax.experimental import pallas as pl
from jax.experimental.pallas import tpu as pltpu
from jax.experimental.pallas import tpu_sc as plsc

_B, _N, _D = 4, 256, 64
_NC, _NS = 2, 16
_NW = _NC * _NS
_ROWS = _B * _N // _NW
_CH = _D // 16


def _precompute_body(pos_ref, vel_ref, mask_ref, idex_ref,
                     dp_ref, rs_ref, m_ref, cnt_ref):
    px = pos_ref[0, :, 0]
    py = pos_ref[0, :, 1]
    dx = px[:, None] - px[None, :]
    dy = py[:, None] - py[None, :]
    dp_ref[0] = jnp.sqrt(dx * dx + dy * dy) + 1e-6
    vx = vel_ref[0, :, 0]
    vy = vel_ref[0, :, 1]
    wx = vx[:, None] - vx[None, :]
    wy = vy[:, None] - vy[None, :]
    rs_ref[0] = jnp.sqrt(wx * wx + wy * wy)
    mask = mask_ref[0]
    m_ref[0] = jnp.where(mask != 0, idex_ref[0], _N)
    cnt_ref[0] = jnp.sum(mask.astype(jnp.float32), axis=-1, keepdims=True)


def _precompute(pos, vel, mask, idex):
    grid = (_B,)
    bs3 = lambda d2: pl.BlockSpec((1, _N, d2), lambda b: (b, 0, 0))
    return pl.pallas_call(
        _precompute_body,
        grid=grid,
        in_specs=[bs3(2), bs3(2), bs3(_N), bs3(_N)],
        out_specs=[bs3(_N), bs3(_N), bs3(_N), bs3(1)],
        out_shape=[
            jax.ShapeDtypeStruct((_B, _N, _N), jnp.float32),
            jax.ShapeDtypeStruct((_B, _N, _N), jnp.float32),
            jax.ShapeDtypeStruct((_B, _N, _N), jnp.int32),
            jax.ShapeDtypeStruct((_B, _N, 1), jnp.float32),
        ],
    )(pos, vel, mask, idex)


def _tbase_body(h_ref, pos_ref, W1_ref, be1_ref, t_ref, base_ref):
    hb = h_ref[0]
    t = jnp.dot(hb, W1_ref[:_D, :], preferred_element_type=jnp.float32)
    px = pos_ref[0, :, 0]
    py = pos_ref[0, :, 1]
    w64 = W1_ref[_D, :]
    w65 = W1_ref[_D + 1, :]
    pw = px[:, None] * w64[None, :] + py[:, None] * w65[None, :]
    t_ref[0] = t + pw
    base_ref[0] = be1_ref[0, :][None, :] - pw


def _tbase(h, pos, We1, be1):
    grid = (_B,)
    return pl.pallas_call(
        _tbase_body,
        grid=grid,
        in_specs=[
            pl.BlockSpec((1, _N, _D), lambda b: (b, 0, 0)),
            pl.BlockSpec((1, _N, 2), lambda b: (b, 0, 0)),
            pl.BlockSpec((_D + 4, _D), lambda b: (0, 0)),
            pl.BlockSpec((1, _D), lambda b: (0, 0)),
        ],
        out_specs=[
            pl.BlockSpec((1, _N, _D), lambda b: (b, 0, 0)),
            pl.BlockSpec((1, _N, _D), lambda b: (b, 0, 0)),
        ],
        out_shape=[
            jax.ShapeDtypeStruct((_B, _N, _D), jnp.float32),
            jax.ShapeDtypeStruct((_B, _N, _D), jnp.float32),
        ],
    )(h, pos, We1, be1.reshape(1, _D))


def _edge_body(t_hbm, base_hbm, m_hbm, dp_hbm, rs_hbm, w_hbm,
               S_hbm,
               t_v, base_v, m_v, dp_v, rs_v, w_v, S_v,
               st_jo, st_d, st_rs):
    wid = lax.axis_index("s") * _NC + lax.axis_index("c")
    r0 = wid * _ROWS
    b = r0 // _N
    pltpu.sync_copy(t_hbm.at[b], t_v.at[pl.ds(0, _N * _D)])
    pltpu.sync_copy(base_hbm.at[pl.ds(r0, _ROWS)], base_v)
    pltpu.sync_copy(m_hbm.at[pl.ds(r0, _ROWS)], m_v)
    pltpu.sync_copy(dp_hbm.at[pl.ds(r0 * _N, _ROWS * _N)],
                    dp_v.at[pl.ds(0, _ROWS * _N)])
    pltpu.sync_copy(rs_hbm.at[pl.ds(r0 * _N, _ROWS * _N)],
                    rs_v.at[pl.ds(0, _ROWS * _N)])
    pltpu.sync_copy(w_hbm, w_v)
    for c in range(_CH):
        t_v[pl.ds(_N * _D + 16 * c, 16)] = jnp.full((16,), -1e30, jnp.float32)
    dp_v[pl.ds(_ROWS * _N, 16)] = jnp.zeros((16,), jnp.float32)
    rs_v[pl.ds(_ROWS * _N, 16)] = jnp.zeros((16,), jnp.float32)

    cols = [lax.iota(jnp.int32, 16) + 16 * c for c in range(_CH)]
    eidx = [jnp.full((16,), e, jnp.int32) for e in range(16)]
    wd = [w_v[0, pl.ds(16 * c, 16)] for c in range(_CH)]
    wrs = [w_v[1, pl.ds(16 * c, 16)] for c in range(_CH)]
    zeros8 = tuple(jnp.zeros((16,), jnp.float32) for _ in range(2 * _CH))

    def row_body(ii, carry):
        bse = [base_v[ii, pl.ds(16 * c, 16)] for c in range(_CH)]
        ii16 = jnp.full((16,), ii * _N, jnp.int32)

        def chunk_body(kc, S):
            k0 = kc * 16
            j16 = m_v[ii, pl.ds(k0, 16)]
            d16 = plsc.load_gather(dp_v, [ii16 + j16])
            rs16 = plsc.load_gather(rs_v, [ii16 + j16])
            st_jo[...] = j16 * _D
            st_d[...] = d16
            st_rs[...] = rs16
            S = list(S)
            for e in range(16):
                jb = plsc.load_gather(st_jo, [eidx[e]])
                db = plsc.load_gather(st_d, [eidx[e]])
                rsb = plsc.load_gather(st_rs, [eidx[e]])
                g = (e & 1) * _CH
                for c in range(_CH):
                    tg = plsc.load_gather(t_v, [jb + cols[c]])
                    a = db * wd[c] + (rsb * wrs[c] + bse[c])
                    S[g + c] = S[g + c] + jnp.maximum(tg + a, 0.0)
            return tuple(S)

        S = lax.fori_loop(0, _N // 16, chunk_body, zeros8)
        for c in range(_CH):
            S_v[ii, pl.ds(16 * c, 16)] = S[c] + S[_CH + c]
        return carry

    lax.fori_loop(0, _ROWS, row_body, 0)
    pltpu.sync_copy(S_v, S_hbm.at[pl.ds(r0, _ROWS)])


def _edge_pass(t_flat, base, m, dpr, rspr, wdrs):
    mesh = plsc.VectorSubcoreMesh(core_axis_name="c", subcore_axis_name="s",
                                  num_cores=_NC, num_subcores=_NS)
    f = pl.kernel(
        _edge_body,
        mesh=mesh,
        compiler_params=pltpu.CompilerParams(use_tc_tiling_on_sc=False,
                                             needs_layout_passes=False),
        out_type=jax.ShapeDtypeStruct((_B * _N, _D), jnp.float32),
        scratch_types=[
            pltpu.VMEM(((_N + 1) * _D,), jnp.float32),
            pltpu.VMEM((_ROWS, _D), jnp.float32),
            pltpu.VMEM((_ROWS, _N), jnp.int32),
            pltpu.VMEM((_ROWS * _N + 16,), jnp.float32),
            pltpu.VMEM((_ROWS * _N + 16,), jnp.float32),
            pltpu.VMEM((2, _D), jnp.float32),
            pltpu.VMEM((_ROWS, _D), jnp.float32),
            pltpu.VMEM((16,), jnp.int32),
            pltpu.VMEM((16,), jnp.float32),
            pltpu.VMEM((16,), jnp.float32),
        ],
    )
    return f(t_flat, base, m, dpr, rspr, wdrs)


def _node_body(h_ref, S_ref, cnt_ref, crowd_ref, W2_ref, be2_ref,
               Wn_ref, bn_ref, gb_ref, out_ref):
    hb = h_ref[0]
    S = S_ref[0]
    cnt = cnt_ref[0]
    agg = jnp.dot(S, W2_ref[...], preferred_element_type=jnp.float32)
    agg = (agg + cnt * be2_ref[0, :][None, :]) / (cnt + 1e-6)
    c = crowd_ref[0]
    mu = jnp.mean(c, axis=-1, keepdims=True)
    var = jnp.mean((c - mu) ** 2, axis=-1, keepdims=True)
    c1 = (c - mu) / jnp.sqrt(var + 1e-5) * gb_ref[0, :][None, :] + gb_ref[1, :][None, :]
    acc = jnp.dot(hb, Wn_ref[:_D, :], preferred_element_type=jnp.float32)
    acc += jnp.dot(agg, Wn_ref[_D:2 * _D, :], preferred_element_type=jnp.float32)
    for cc in range(5):
        acc += c1[:, cc:cc + 1] * Wn_ref[2 * _D + cc, :][None, :]
    out_ref[0] = jnp.maximum(acc + bn_ref[0, :][None, :], 0.0)


def _node_update(h, S, cnt, crowd, p):
    grid = (_B,)
    crowd_dim = crowd.shape[-1]
    gb = jnp.stack([p['g'], p['b']], axis=0)
    return pl.pallas_call(
        _node_body,
        grid=grid,
        in_specs=[
            pl.BlockSpec((1, _N, _D), lambda b: (b, 0, 0)),
            pl.BlockSpec((1, _N, _D), lambda b: (b, 0, 0)),
            pl.BlockSpec((1, _N, 1), lambda b: (b, 0, 0)),
            pl.BlockSpec((1, _N, crowd_dim), lambda b: (b, 0, 0)),
            pl.BlockSpec((_D, _D), lambda b: (0, 0)),
            pl.BlockSpec((1, _D), lambda b: (0, 0)),
            pl.BlockSpec((2 * _D + crowd_dim, _D), lambda b: (0, 0)),
            pl.BlockSpec((1, _D), lambda b: (0, 0)),
            pl.BlockSpec((2, crowd_dim), lambda b: (0, 0)),
        ],
        out_specs=pl.BlockSpec((1, _N, _D), lambda b: (b, 0, 0)),
        out_shape=jax.ShapeDtypeStruct((_B, _N, _D), jnp.float32),
    )(h, S, cnt, crowd, p['We2'], p['be2'].reshape(1, _D),
      p['Wn'], p['bn'].reshape(1, _D), gb)


def kernel(h, pos, vel, acc, crowd, mask, idex, hist, params):
    del acc, hist
    dpair, rspair, m, cnt = _precompute(pos, vel, mask, idex)
    m2 = m.reshape(_B * _N, _N)
    dp2 = dpair.reshape(_B * _N * _N)
    rs2 = rspair.reshape(_B * _N * _N)
    x = h
    for p in params:
        t, base = _tbase(x, pos, p['We1'], p['be1'])
        wdrs = jnp.stack([p['We1'][_D + 2], p['We1'][_D + 3]], axis=0)
        S = _edge_pass(t.reshape(_B, _N * _D), base.reshape(_B * _N, _D),
                       m2, dp2, rs2, wdrs)
        x = _node_update(x, S.reshape(_B, _N, _D), cnt, crowd, p)
    return x
